# baseline (speedup 1.0000x reference)
import jax
import jax.numpy as jnp
from jax.experimental import pallas as pl
from jax.experimental.pallas import tpu as pltpu


def _gc_kernel(adj_ref, x_ref, w_ref, out_ref):
    BB = adj_ref.shape[0]
    for b in range(BB):
        xw = jnp.dot(x_ref[b], w_ref[...], preferred_element_type=jnp.float32)
        y = jnp.dot(adj_ref[b], xw, preferred_element_type=jnp.float32)
        out_ref[b] = y.astype(out_ref.dtype)


def kernel(adjacency, nodes, weights):
    B, W, N, _ = adjacency.shape
    Fin = nodes.shape[-1]
    Fout = weights.shape[-1]
    itemsize = jnp.dtype(adjacency.dtype).itemsize

    flops = 2 * B * W * (N * N * Fout + N * Fin * Fout)
    bytes_accessed = itemsize * (adjacency.size + nodes.size + weights.size
                                 + B * W * N * Fout)
    cost = pl.CostEstimate(flops=flops, transcendentals=0,
                           bytes_accessed=bytes_accessed)

    BB = B // 2
    return pl.pallas_call(
        _gc_kernel,
        out_shape=jax.ShapeDtypeStruct((B, W, N, Fout), nodes.dtype),
        grid_spec=pl.GridSpec(
            grid=(2, W),
            in_specs=[
                pl.BlockSpec((BB, pl.Squeezed(), N, N),
                             lambda i, w: (i, w, 0, 0)),
                pl.BlockSpec((BB, pl.Squeezed(), N, Fin),
                             lambda i, w: (i, w, 0, 0)),
                pl.BlockSpec((pl.Squeezed(), Fin, Fout),
                             lambda i, w: (w, 0, 0)),
            ],
            out_specs=pl.BlockSpec((BB, pl.Squeezed(), N, Fout),
                                   lambda i, w: (i, w, 0, 0)),
        ),
        compiler_params=pltpu.CompilerParams(
            dimension_semantics=("parallel", "arbitrary"),
            vmem_limit_bytes=48 * 1024 * 1024,
        ),
        cost_estimate=cost,
    )(adjacency, nodes, weights)


# speedup vs baseline: 1.0093x; 1.0093x over previous
import functools

import jax
import jax.numpy as jnp
from jax.experimental import pallas as pl
from jax.experimental.pallas import tpu as pltpu

_CHUNKS = ((0, 1), (1, 2), (2, 4), (4, 8))


def _gc_kernel_body(adj_hbm, x_hbm, w_hbm, out_hbm,
                    x_buf, w_buf, adj_buf, o_buf,
                    adj_sem, x_sem, w_sem, out_sem,
                    *, W, npairs):
    i = pl.program_id(0)
    p0 = i * npairs

    def chunk_copy(c):
        s, e = _CHUNKS[c]
        return pltpu.make_async_copy(
            adj_hbm.at[pl.ds(p0 + s, e - s)], adj_buf.at[pl.ds(s, e - s)],
            adj_sem.at[c])

    def out_copy(k):
        return pltpu.make_async_copy(
            o_buf.at[k], out_hbm.at[p0 + k], out_sem.at[k])

    x_copy = pltpu.make_async_copy(x_hbm.at[pl.ds(p0, npairs)], x_buf, x_sem)
    w_copy = pltpu.make_async_copy(w_hbm, w_buf, w_sem)

    chunk_copy(0).start()
    x_copy.start()
    w_copy.start()
    for c in range(1, len(_CHUNKS)):
        chunk_copy(c).start()
    x_copy.wait()
    w_copy.wait()

    chunk_of_pair = []
    for c, (s, e) in enumerate(_CHUNKS):
        chunk_of_pair += [c] * (e - s)

    waited = set()
    for k in range(npairs):
        c = chunk_of_pair[k]
        if c not in waited:
            chunk_copy(c).wait()
            waited.add(c)
        xw = jnp.dot(x_buf[k], w_buf[k % W],
                     preferred_element_type=jnp.float32)
        o_buf[k] = jnp.dot(adj_buf[k], xw,
                           preferred_element_type=jnp.float32)
        out_copy(k).start()

    for k in range(npairs):
        out_copy(k).wait()


def kernel(adjacency, nodes, weights):
    B, W, N, _ = adjacency.shape
    Fin = nodes.shape[-1]
    Fout = weights.shape[-1]
    itemsize = jnp.dtype(adjacency.dtype).itemsize
    npairs = (B * W) // 2

    flops = 2 * B * W * (N * N * Fout + N * Fin * Fout)
    bytes_accessed = itemsize * (adjacency.size + nodes.size + weights.size
                                 + B * W * N * Fout)
    cost = pl.CostEstimate(flops=flops, transcendentals=0,
                           bytes_accessed=bytes_accessed)

    body = functools.partial(_gc_kernel_body, W=W, npairs=npairs)

    out_flat = pl.pallas_call(
        body,
        out_shape=jax.ShapeDtypeStruct((B * W, N, Fout), nodes.dtype),
        grid=(2,),
        in_specs=[
            pl.BlockSpec(memory_space=pl.ANY),
            pl.BlockSpec(memory_space=pl.ANY),
            pl.BlockSpec(memory_space=pl.ANY),
        ],
        out_specs=pl.BlockSpec(memory_space=pl.ANY),
        scratch_shapes=[
            pltpu.VMEM((npairs, N, Fin), jnp.float32),
            pltpu.VMEM((W, Fin, Fout), jnp.float32),
            pltpu.VMEM((npairs, N, N), jnp.float32),
            pltpu.VMEM((npairs, N, Fout), jnp.float32),
            pltpu.SemaphoreType.DMA((len(_CHUNKS),)),
            pltpu.SemaphoreType.DMA,
            pltpu.SemaphoreType.DMA,
            pltpu.SemaphoreType.DMA((npairs,)),
        ],
        compiler_params=pltpu.CompilerParams(
            dimension_semantics=("parallel",),
            vmem_limit_bytes=48 * 1024 * 1024,
        ),
        cost_estimate=cost,
    )(adjacency.reshape(B * W, N, N), nodes.reshape(B * W, N, Fin), weights)

    return out_flat.reshape(B, W, N, Fout)
